# Initial kernel scaffold; baseline (speedup 1.0000x reference)
import jax
import jax.numpy as jnp
from jax.experimental import pallas as pl


def kernel(x, adj_t, Wl0, bl0, Wr0, Wl1, bl1, Wr1):
    raise NotImplementedError("write your pallas kernel here")


# speedup vs baseline: 2.9580x; 2.9580x over previous
import functools

import jax
import jax.numpy as jnp
from jax import lax
from jax.experimental import pallas as pl
from jax.experimental.pallas import tpu as pltpu
from jax.experimental.pallas import tpu_sc as plsc

D = 128
W_AUG = 144
NC, NS, LANES = 2, 16, 16
NW = NC * NS
BATCH = 128
STAGE = 8


def _agg_body(n_nodes, n_pad, table, src2d, dst2d, out,
              idx_s, idx_d, rows, sem, acc):
    cid = lax.axis_index("c")
    sid = lax.axis_index("s")
    wid = sid * NC + cid

    def zr(r, carry):
        for cc in range(W_AUG // LANES):
            rows[r, pl.ds(cc * LANES, LANES)] = jnp.zeros((LANES,), jnp.float32)
        return carry
    lax.fori_loop(0, BATCH, zr, 0)
    zchunks = n_pad // NS // BATCH
    for k in range(zchunks):
        pltpu.sync_copy(rows, acc.at[pl.ds((sid * zchunks + k) * BATCH, BATCH)])
    plsc.subcore_barrier()

    n_idx_rows = src2d.shape[0]
    rows_per_w = n_idx_rows // NW
    row_base = wid * rows_per_w
    nstage = rows_per_w // STAGE

    def stage_body(t, carry):
        roff = row_base + t * STAGE
        pltpu.sync_copy(src2d.at[pl.ds(roff, STAGE)], idx_s)
        pltpu.sync_copy(dst2d.at[pl.ds(roff, STAGE)], idx_d)
        for j in range(STAGE):
            pltpu.async_copy(table.at[idx_s.at[j]], rows, sem).wait()
            pltpu.sync_copy(rows, acc.at[idx_d.at[j]], add=True)
        return carry
    lax.fori_loop(0, nstage, stage_body, 0)
    plsc.subcore_barrier()

    rpw = n_pad // NS
    pltpu.sync_copy(acc.at[pl.ds(sid * rpw, rpw)],
                    out.at[cid, pl.ds(sid * rpw, rpw)])


def _make_agg(n_nodes, n_pad):
    mesh = plsc.VectorSubcoreMesh(core_axis_name="c", subcore_axis_name="s")
    return pl.kernel(
        functools.partial(_agg_body, n_nodes, n_pad),
        out_type=jax.ShapeDtypeStruct((NC, n_pad, W_AUG), jnp.float32),
        mesh=mesh,
        compiler_params=pltpu.CompilerParams(use_tc_tiling_on_sc=False),
        scratch_types=[
            pltpu.VMEM((STAGE, BATCH), jnp.int32),
            pltpu.VMEM((STAGE, BATCH), jnp.int32),
            pltpu.VMEM((BATCH, W_AUG), jnp.float32),
            pltpu.SemaphoreType.DMA,
            pltpu.VMEM_SHARED((n_pad, W_AUG), jnp.float32),
        ],
    )


def _tc_layer(p0, p1, root, wl, bl, wr, make_next):
    n = root.shape[0]
    rb = 400
    grid = (n // rb,)
    rw = root.shape[1]

    def body(p0_ref, p1_ref, x_ref, wl_ref, bl_ref, wr_ref, o1_ref, *rest):
        s = p0_ref[...] + p1_ref[...]
        cnt = s[:, D:D + 1]
        mean = s[:, :D] / jnp.maximum(cnt, 1.0)
        xr = x_ref[...][:, :D]
        h1 = (jnp.dot(mean, wl_ref[...], preferred_element_type=jnp.float32)
              + bl_ref[...]
              + jnp.dot(xr, wr_ref[...], preferred_element_type=jnp.float32))
        o1_ref[...] = h1
        if make_next:
            h = jnp.maximum(h1, 0.0)
            aug = jnp.pad(h, ((0, 0), (0, W_AUG - D)))
            col = lax.broadcasted_iota(jnp.int32, (rb, W_AUG), 1)
            rest[0][...] = jnp.where(col == D, 1.0, aug)

    out_shape = [jax.ShapeDtypeStruct((n, D), jnp.float32)]
    if make_next:
        out_shape.append(jax.ShapeDtypeStruct((n, W_AUG), jnp.float32))
    outs = pl.pallas_call(
        body,
        grid=grid,
        in_specs=[
            pl.BlockSpec((rb, W_AUG), lambda i: (i, 0)),
            pl.BlockSpec((rb, W_AUG), lambda i: (i, 0)),
            pl.BlockSpec((rb, rw), lambda i: (i, 0)),
            pl.BlockSpec((D, D), lambda i: (0, 0)),
            pl.BlockSpec((1, D), lambda i: (0, 0)),
            pl.BlockSpec((D, D), lambda i: (0, 0)),
        ],
        out_specs=[pl.BlockSpec((rb, D), lambda i: (i, 0))]
        + ([pl.BlockSpec((rb, W_AUG), lambda i: (i, 0))] if make_next else []),
        out_shape=out_shape,
    )(p0, p1, root, wl, bl.reshape(1, D), wr)
    return outs


def kernel(x, adj_t, Wl0, bl0, Wr0, Wl1, bl1, Wr1):
    n = x.shape[0]
    src = adj_t[0].astype(jnp.int32)
    dst = adj_t[1].astype(jnp.int32)
    e = src.shape[0]

    unit = NW * BATCH * STAGE
    e_pad = ((e + unit - 1) // unit) * unit
    pad = e_pad - e
    src_p = jnp.concatenate([src, jnp.zeros((pad,), jnp.int32)])
    dst_p = jnp.concatenate([dst, jnp.full((pad,), n, jnp.int32)])
    src2d = src_p.reshape(e_pad // BATCH, BATCH)
    dst2d = dst_p.reshape(e_pad // BATCH, BATCH)

    zunit = NS * BATCH
    n_pad = ((n + 1 + zunit - 1) // zunit) * zunit

    agg = _make_agg(n, n_pad)

    x_aug = jnp.concatenate(
        [x, jnp.ones((n, 1), jnp.float32), jnp.zeros((n, W_AUG - D - 1), jnp.float32)],
        axis=1)

    p = agg(x_aug, src2d, dst2d)
    h1, h_aug = _tc_layer(p[0], p[1], x, Wl0, bl0, Wr0, make_next=True)
    p2 = agg(h_aug, src2d, dst2d)
    h2 = _tc_layer(p2[0], p2[1], h_aug, Wl1, bl1, Wr1, make_next=False)[0]
    return (h1, h2)
